# baseline (speedup 1.0000x reference)
import functools

import jax
import jax.numpy as jnp
from jax import lax
from jax.experimental import pallas as pl
from jax.experimental.pallas import tpu as pltpu

_F32 = jnp.float32


def _l2norm(h):
    sumsq = jnp.sum(h * h, axis=-1, keepdims=True)
    return h * lax.rsqrt(jnp.maximum(sumsq, 1e-24))


def _fused_kernel(nb1, nb2, blk1, num_snapshots, dim_inner,
                  x_ref, af_ref, as_ref,
                  wpre_ref, bpre_ref, wmp0_ref, bmp0_ref,
                  wmp1_ref, b1_ref, whead_ref, bhead_ref,
                  o_ref, t_ref, u_ref, w1_ref):
    i = pl.program_id(0)
    S, D = num_snapshots, dim_inner

    @pl.when(i == 0)
    def _init():
        h = jnp.dot(x_ref[...], wpre_ref[...],
                    preferred_element_type=_F32) + bpre_ref[...]
        h = jnp.maximum(h, 0.0)
        t_ref[...] = jnp.dot(h, wmp0_ref[...], preferred_element_type=_F32)
        w1_ref[...] = jnp.concatenate(
            [wmp1_ref[s] for s in range(S)], axis=1)

    @pl.when(i < nb1)
    def _phase1():
        h1 = jnp.dot(af_ref[...], t_ref[...],
                     preferred_element_type=_F32) + bmp0_ref[...]
        h1 = _l2norm(jnp.maximum(h1, 0.0))
        u_ref[pl.ds(i * blk1, blk1), :] = jnp.dot(
            h1, w1_ref[...], preferred_element_type=_F32)

    @pl.when(i >= nb1)
    def _phase2():
        acc = jnp.dot(as_ref[0], u_ref[:, 0:D], preferred_element_type=_F32)
        for s in range(1, S):
            acc = acc + jnp.dot(as_ref[s], u_ref[:, s * D:(s + 1) * D],
                                preferred_element_type=_F32)
        bsum = jnp.sum(b1_ref[...], axis=0)
        hf = _l2norm(jnp.maximum(acc + bsum, 0.0))
        out = jnp.dot(hf, whead_ref[...],
                      preferred_element_type=_F32) + bhead_ref[...]
        o_ref[...] = out.astype(o_ref.dtype)


def kernel(x, adj_full, adj_snapshots,
           w_pre, b_pre, w_mp0, b_mp0, w_mp1, b_mp1, w_head, b_head):
    N, dim_in = x.shape
    S = adj_snapshots.shape[0]
    dim_inner = w_pre.shape[1]
    dim_out = w_head.shape[1]

    blk1 = 128 if N % 128 == 0 and N > 128 else N
    blk2 = 128 if N % 128 == 0 and N > 128 else N
    nb1 = N // blk1
    nb2 = N // blk2

    bpre = b_pre.reshape(1, dim_inner)
    bmp0 = b_mp0.reshape(1, dim_inner)
    b1 = b_mp1.reshape(S, 1, dim_inner)
    bhead = b_head.reshape(1, dim_out)

    def af_idx(i):
        return (jnp.minimum(i, nb1 - 1), 0)

    def as_idx(i):
        return (0, jnp.clip(i - nb1, 0, nb2 - 1), 0)

    def out_idx(i):
        return (jnp.clip(i - nb1, 0, nb2 - 1), 0)

    return pl.pallas_call(
        functools.partial(_fused_kernel, nb1, nb2, blk1, S, dim_inner),
        out_shape=jax.ShapeDtypeStruct((N, dim_out), x.dtype),
        grid=(nb1 + nb2,),
        in_specs=[
            pl.BlockSpec((N, dim_in), lambda i: (0, 0)),
            pl.BlockSpec((blk1, N), af_idx),
            pl.BlockSpec((S, blk2, N), as_idx),
            pl.BlockSpec((dim_in, dim_inner), lambda i: (0, 0)),
            pl.BlockSpec((1, dim_inner), lambda i: (0, 0)),
            pl.BlockSpec((dim_inner, dim_inner), lambda i: (0, 0)),
            pl.BlockSpec((1, dim_inner), lambda i: (0, 0)),
            pl.BlockSpec((S, dim_inner, dim_inner), lambda i: (0, 0, 0)),
            pl.BlockSpec((S, 1, dim_inner), lambda i: (0, 0, 0)),
            pl.BlockSpec((dim_inner, dim_out), lambda i: (0, 0)),
            pl.BlockSpec((1, dim_out), lambda i: (0, 0)),
        ],
        out_specs=pl.BlockSpec((blk2, dim_out), out_idx),
        scratch_shapes=[
            pltpu.VMEM((N, dim_inner), _F32),
            pltpu.VMEM((N, S * dim_inner), _F32),
            pltpu.VMEM((dim_inner, S * dim_inner), _F32),
        ],
        compiler_params=pltpu.CompilerParams(
            dimension_semantics=("arbitrary",)),
    )(x, adj_full, adj_snapshots,
      w_pre, bpre, w_mp0, bmp0, w_mp1, b1, w_head, bhead)
